# baseline (speedup 1.0000x reference)
import functools

import jax
import jax.numpy as jnp
from jax import lax
from jax.experimental import pallas as pl
from jax.experimental.pallas import tpu as pltpu
from jax.experimental.pallas import tpu_sc as plsc

_NUM_CORES = 2
_NUM_SUBCORES = 16
_LANES = 128
_ALIGN = 1024
_PIPE_LAG = 8
_Q_SPLIT = 192
_QB = 8


def _sc_body(q_len, v_len, dim, max_pos, region_rows,
             big_hbm, out_hbm, table_v, sem):
  cid = lax.axis_index("c")
  sid = lax.axis_index("s")

  pltpu.sync_copy(big_hbm.at[pl.ds(sid * region_rows, region_rows), :],
                  table_v)

  row_rows = v_len * dim // _LANES
  pad = lax.rem((16 - sid) * 64, _ALIGN)
  q_lo = lax.rem(max_pos - sid, 16)

  copies = []
  for k in range(_Q_SPLIT // 16):
    q_row = q_lo + 16 * k
    s = (max_pos - q_row) * dim
    src_row = lax.div(pad + s, _LANES)
    for bb in range(2):
      r = (cid * 2 + bb) * q_len + q_row
      c = pltpu.make_async_copy(
          table_v.at[pl.ds(src_row, row_rows), :],
          out_hbm.at[pl.ds(r * row_rows, row_rows), :],
          sem,
      )
      c.start()
      copies.append(c)
      if len(copies) > _PIPE_LAG:
        copies[len(copies) - 1 - _PIPE_LAG].wait()
  for c in copies[-_PIPE_LAG:]:
    c.wait()


def _tc_body(q_len, v_len, dim, max_pos, region_words,
             big_ref, sc_ref, out_ref):
  del sc_ref
  j = pl.program_id(1)
  row_rows = v_len * dim // _LANES
  for r in range(_QB):
    q_row = _Q_SPLIT + j * _QB + r
    s = (max_pos - q_row) * dim
    jj = lax.rem(lax.div(s, 64), 16)
    pad = lax.rem((16 - jj) * 64, _ALIGN)
    src_row = lax.div(jj * region_words + pad + s, _LANES)
    out_ref[pl.ds(r * row_rows, row_rows), :] = (
        big_ref[pl.ds(src_row, row_rows), :])


def kernel(q, v, embeddings):
  batch, q_len = q.shape[0], q.shape[1]
  v_len = v.shape[1]
  table_rows, dim = embeddings.shape
  max_pos = (table_rows - 1) // 2

  assert batch == 2 * _NUM_CORES and q_len % _NUM_SUBCORES == 0
  assert (v_len * dim) % _LANES == 0

  table_words = table_rows * dim
  region_words = -(-(960 + table_words) // _ALIGN) * _ALIGN
  region_rows = region_words // _LANES

  flat = embeddings.reshape(-1)
  big = jnp.zeros((16 * region_words,), jnp.float32)
  for j in range(16):
    pad = (16 - j) * 64 % _ALIGN
    big = lax.dynamic_update_slice(big, flat, (j * region_words + pad,))
  big2d = big.reshape(-1, _LANES)
  n_big_rows = big2d.shape[0]

  n_rows = batch * q_len
  row_rows = v_len * dim // _LANES

  mesh = plsc.VectorSubcoreMesh(core_axis_name="c", subcore_axis_name="s")
  sc_fn = functools.partial(
      _sc_body, q_len, v_len, dim, max_pos, region_rows)
  sc_out = pl.kernel(
      sc_fn,
      out_type=jax.ShapeDtypeStruct((n_rows * row_rows, _LANES),
                                    jnp.float32),
      mesh=mesh,
      scratch_types=[
          pltpu.VMEM((region_rows, _LANES), jnp.float32),
          pltpu.SemaphoreType.DMA,
      ],
  )(big2d)

  tc_fn = functools.partial(
      _tc_body, q_len, v_len, dim, max_pos, region_words)
  blk_rows = _QB * row_rows
  out = pl.pallas_call(
      tc_fn,
      grid=(batch, (q_len - _Q_SPLIT) // _QB),
      in_specs=[
          pl.BlockSpec((n_big_rows, _LANES), lambda b, j: (0, 0)),
          pl.BlockSpec(memory_space=pl.ANY),
      ],
      out_specs=pl.BlockSpec(
          (blk_rows, _LANES),
          lambda b, j: ((b * q_len + _Q_SPLIT) // _QB + j, 0)),
      out_shape=jax.ShapeDtypeStruct((n_rows * row_rows, _LANES),
                                     jnp.float32),
      input_output_aliases={1: 0},
  )(big2d, sc_out)
  return out.reshape(batch, q_len, v_len, dim)


# speedup vs baseline: 1.0510x; 1.0510x over previous
import functools

import jax
import jax.numpy as jnp
from jax import lax
from jax.experimental import pallas as pl
from jax.experimental.pallas import tpu as pltpu
from jax.experimental.pallas import tpu_sc as plsc

_NUM_CORES = 2
_NUM_SUBCORES = 16
_LANES = 128
_ALIGN = 1024
_PIPE_LAG = 16


def _rel_pos_body(n_q_per_tile, q_len, v_len, dim, max_pos, region_rows,
                  big_hbm, out_hbm, table_v, sem):
  cid = lax.axis_index("c")
  sid = lax.axis_index("s")

  pltpu.sync_copy(big_hbm.at[pl.ds(sid * region_rows, region_rows), :],
                  table_v)

  row_words = v_len * dim
  row_rows = row_words // _LANES
  pad = lax.rem((16 - sid) * 64, _ALIGN)
  q_lo = lax.rem(max_pos - sid, 16)

  copies = []
  for k in range(n_q_per_tile):
    q_row = q_lo + 16 * k
    s = (max_pos - q_row) * dim
    src_row = lax.div(pad + s, _LANES)
    for bb in range(2):
      b = cid * 2 + bb
      r = b * q_len + q_row
      c = pltpu.make_async_copy(
          table_v.at[pl.ds(src_row, row_rows), :],
          out_hbm.at[pl.ds(r * row_rows, row_rows), :],
          sem,
      )
      c.start()
      copies.append(c)
      if len(copies) > _PIPE_LAG:
        copies[len(copies) - 1 - _PIPE_LAG].wait()
  for c in copies[-_PIPE_LAG:]:
    c.wait()


def kernel(q, v, embeddings):
  batch, q_len = q.shape[0], q.shape[1]
  v_len = v.shape[1]
  table_rows, dim = embeddings.shape
  max_pos = (table_rows - 1) // 2

  assert batch == 2 * _NUM_CORES and q_len % _NUM_SUBCORES == 0
  n_q_per_tile = q_len // _NUM_SUBCORES
  assert (v_len * dim) % _LANES == 0

  table_words = table_rows * dim
  region_words = -(-(960 + table_words) // _ALIGN) * _ALIGN
  region_rows = region_words // _LANES

  flat = embeddings.reshape(-1)
  big = jnp.zeros((16 * region_words,), jnp.float32)
  for j in range(16):
    pad = (16 - j) * 64 % _ALIGN
    big = lax.dynamic_update_slice(big, flat, (j * region_words + pad,))
  big2d = big.reshape(-1, _LANES)

  mesh = plsc.VectorSubcoreMesh(core_axis_name="c", subcore_axis_name="s")
  body = functools.partial(
      _rel_pos_body, n_q_per_tile, q_len, v_len, dim, max_pos, region_rows)

  n_rows = batch * q_len
  run = pl.kernel(
      body,
      out_type=jax.ShapeDtypeStruct((n_rows * v_len * dim // _LANES, _LANES),
                                    jnp.float32),
      mesh=mesh,
      scratch_types=[
          pltpu.VMEM((region_rows, _LANES), jnp.float32),
          pltpu.SemaphoreType.DMA,
      ],
  )
  out = run(big2d)
  return out.reshape(batch, q_len, v_len, dim)
